# baseline (speedup 1.0000x reference)
import functools
import math

import jax
import jax.numpy as jnp
import numpy as np
from jax import lax
from jax.experimental import pallas as pl
from jax.experimental.pallas import tpu as pltpu
from jax.experimental.pallas import tpu_sc as plsc

D = 128
SEQ = 200
UNIT = 80
NBUF = 8
SCALE = math.sqrt(float(D))


def _positional_encoding(length, depth):
    half = depth // 2
    positions = np.arange(length)[:, None].astype(np.float32)
    depths = np.arange(half)[None, :].astype(np.float32) / float(half)
    angle_rates = 1.0 / (10000.0 ** depths)
    angle_rads = positions * angle_rates
    return np.concatenate([np.sin(angle_rads), np.cos(angle_rads)], axis=-1)


_PE = jnp.asarray(_positional_encoding(2048, D)[:SEQ], dtype=jnp.float32)


@functools.cache
def _make_kernel(n_tokens):
    info = plsc.get_sparse_core_info()
    nc, ns = info.num_cores, info.num_subcores
    nw = nc * ns
    upw = n_tokens // (nw * UNIT)
    n_pairs = upw // (2 * NBUF // 2)
    ss = NBUF // 2
    mesh = plsc.VectorSubcoreMesh(core_axis_name="c", subcore_axis_name="s")

    @functools.partial(
        pl.kernel,
        out_type=jax.ShapeDtypeStruct((n_tokens // UNIT, UNIT, D),
                                      jnp.float32),
        mesh=mesh,
        scratch_types=[
            pltpu.VMEM((upw * UNIT,), jnp.int32),
            pltpu.VMEM((SEQ, D), jnp.float32),
        ] + [pltpu.VMEM((UNIT, D), jnp.float32)] * NBUF
          + [pltpu.SemaphoreType.DMA] * (2 * NBUF),
    )
    def k(x_hbm, table_hbm, pe_hbm, out_hbm, idx_v, pe_v,
          r0, r1, r2, r3, r4, r5, r6, r7,
          g0, g1, g2, g3, g4, g5, g6, g7,
          o0, o1, o2, o3, o4, o5, o6, o7):
        rows = (r0, r1, r2, r3, r4, r5, r6, r7)
        gsem = (g0, g1, g2, g3, g4, g5, g6, g7)
        osem = (o0, o1, o2, o3, o4, o5, o6, o7)
        wid = lax.axis_index("s") * nc + lax.axis_index("c")
        pltpu.sync_copy(x_hbm.at[pl.ds(wid * upw * UNIT, upw * UNIT)],
                        idx_v)
        pltpu.sync_copy(pe_hbm, pe_v)

        def gather(u, b):
            return pltpu.make_async_copy(
                table_hbm.at[idx_v.at[pl.ds(u * UNIT, UNIT)]], rows[b],
                gsem[b])

        def out_cp(u, b):
            return pltpu.make_async_copy(
                rows[b], out_hbm.at[wid * upw + u], osem[b])

        def compute(b, u):
            buf = rows[b]
            ng = D // 16
            po = lax.rem(u * UNIT, SEQ)

            def row_body(t, c):
                embs = [buf[t, pl.ds(g * 16, 16)] for g in range(ng)]
                pes = [pe_v[c, pl.ds(g * 16, 16)] for g in range(ng)]
                for g in range(ng):
                    buf[t, pl.ds(g * 16, 16)] = embs[g] * SCALE + pes[g]
                return lax.select(c + 1 == SEQ, 0, c + 1)

            lax.fori_loop(0, UNIT, row_body, po)

        for j in range(ss):
            gather(j, j).start()

        def pair(p, carry):
            uA = 2 * ss * p
            uB = 2 * ss * p + ss
            for j in range(ss):
                gather(uA + j, j).wait()
                compute(j, uA + j)
            for j in range(ss):
                out_cp(uA + j, j).start()

            @pl.when(p >= 1)
            def _():
                for j in range(ss):
                    out_cp(uA - ss + j, ss + j).wait()

            for j in range(ss):
                gather(uB + j, ss + j).start()
            for j in range(ss):
                gather(uB + j, ss + j).wait()
                compute(ss + j, uB + j)
            for j in range(ss):
                out_cp(uB + j, ss + j).start()
            for j in range(ss):
                out_cp(uA + j, j).wait()

            @pl.when(p + 1 < n_pairs)
            def _():
                for j in range(ss):
                    gather(uB + ss + j, j).start()

            return carry

        lax.fori_loop(0, n_pairs, pair, 0)
        for j in range(ss):
            out_cp(upw - ss + j, ss + j).wait()

    return k


def kernel(x, table):
    n_batch, seq = x.shape
    n_tokens = n_batch * seq
    out = _make_kernel(n_tokens)(x.reshape(-1), table, _PE)
    return out.reshape(n_batch, seq, D)


# speedup vs baseline: 1.0577x; 1.0577x over previous
import functools
import math

import jax
import jax.numpy as jnp
import numpy as np
from jax import lax
from jax.experimental import pallas as pl
from jax.experimental.pallas import tpu as pltpu
from jax.experimental.pallas import tpu_sc as plsc

D = 128
SEQ = 200
UNIT = 200
NBUF = 4
SCALE = math.sqrt(float(D))


def _positional_encoding(length, depth):
    half = depth // 2
    positions = np.arange(length)[:, None].astype(np.float32)
    depths = np.arange(half)[None, :].astype(np.float32) / float(half)
    angle_rates = 1.0 / (10000.0 ** depths)
    angle_rads = positions * angle_rates
    return np.concatenate([np.sin(angle_rads), np.cos(angle_rads)], axis=-1)


_PE = jnp.asarray(_positional_encoding(2048, D)[:SEQ], dtype=jnp.float32)


@functools.cache
def _make_kernel(n_tokens):
    info = plsc.get_sparse_core_info()
    nc, ns = info.num_cores, info.num_subcores
    nw = nc * ns
    upw = n_tokens // (nw * UNIT)
    n_pairs = upw // 4
    mesh = plsc.VectorSubcoreMesh(core_axis_name="c", subcore_axis_name="s")

    @functools.partial(
        pl.kernel,
        out_type=jax.ShapeDtypeStruct((n_tokens // UNIT, UNIT, D),
                                      jnp.float32),
        mesh=mesh,
        scratch_types=[
            pltpu.VMEM((SEQ, D), jnp.float32),
        ] + [pltpu.VMEM((UNIT, D), jnp.float32)] * NBUF
          + [pltpu.VMEM((UNIT,), jnp.int32)] * NBUF
          + [pltpu.SemaphoreType.DMA] * (3 * NBUF),
    )
    def k(x_hbm, table_hbm, pe_hbm, out_hbm, pe_v,
          r0, r1, r2, r3, x0, x1, x2, x3,
          g0, g1, g2, g3, o0, o1, o2, o3, i0, i1, i2, i3):
        rows = (r0, r1, r2, r3)
        idxs = (x0, x1, x2, x3)
        gsem = (g0, g1, g2, g3)
        osem = (o0, o1, o2, o3)
        isem = (i0, i1, i2, i3)
        wid = lax.axis_index("s") * nc + lax.axis_index("c")
        pltpu.sync_copy(pe_hbm, pe_v)
        tok_base = wid * upw * UNIT

        def idx_cp(u, b):
            return pltpu.make_async_copy(
                x_hbm.at[pl.ds(tok_base + u * UNIT, UNIT)], idxs[b],
                isem[b])

        def gather(b):
            return pltpu.make_async_copy(
                table_hbm.at[idxs[b]], rows[b], gsem[b])

        def out_cp(u, b):
            return pltpu.make_async_copy(
                rows[b], out_hbm.at[wid * upw + u], osem[b])

        def compute(b):
            buf = rows[b]
            ng = D // 16

            def row_body(t, c):
                embs = [buf[t, pl.ds(g * 16, 16)] for g in range(ng)]
                pes = [pe_v[t, pl.ds(g * 16, 16)] for g in range(ng)]
                for g in range(ng):
                    buf[t, pl.ds(g * 16, 16)] = embs[g] * SCALE + pes[g]
                return c

            lax.fori_loop(0, UNIT, row_body, 0)

        for j in range(NBUF):
            idx_cp(j, j).start()
        for j in range(2):
            idx_cp(j, j).wait()
            gather(j).start()

        def pair(p, carry):
            uA = 4 * p
            uB = 4 * p + 2
            for j in range(2):
                gather(j).wait()
                compute(j)

            @pl.when(p + 1 < n_pairs)
            def _():
                for j in range(2):
                    idx_cp(uA + 4 + j, j).start()

            for j in range(2):
                out_cp(uA + j, j).start()

            @pl.when(p >= 1)
            def _():
                for j in range(2):
                    out_cp(uA - 2 + j, 2 + j).wait()

            for j in range(2):
                idx_cp(uB + j, 2 + j).wait()
                gather(2 + j).start()
            for j in range(2):
                gather(2 + j).wait()
                compute(2 + j)

            @pl.when(p + 1 < n_pairs)
            def _():
                for j in range(2):
                    idx_cp(uB + 4 + j, 2 + j).start()

            for j in range(2):
                out_cp(uB + j, 2 + j).start()
            for j in range(2):
                out_cp(uA + j, j).wait()

            @pl.when(p + 1 < n_pairs)
            def _():
                for j in range(2):
                    idx_cp(uA + 4 + j, j).wait()
                    gather(j).start()

            return carry

        lax.fori_loop(0, n_pairs, pair, 0)
        for j in range(2):
            out_cp(upw - 2 + j, 2 + j).wait()

    return k


def kernel(x, table):
    n_batch, seq = x.shape
    n_tokens = n_batch * seq
    out = _make_kernel(n_tokens)(x.reshape(-1), table, _PE)
    return out.reshape(n_batch, seq, D)
